# Initial kernel scaffold; baseline (speedup 1.0000x reference)
import jax
import jax.numpy as jnp
from jax.experimental import pallas as pl


def kernel(hidden_states, typo_vecs, W1, b1, W2, b2, We, be):
    raise NotImplementedError("write your pallas kernel here")


# speedup vs baseline: 20.1091x; 20.1091x over previous
import jax
import jax.numpy as jnp
from jax.experimental import pallas as pl

_B, _S, _H = 4, 2048, 768
_T = 65
_E = 8
_L = 128
_DR = 256
_N = _B * _S
_BLK = 512


def _moe_block(ri_ref, W1_ref, b1_ref, W2_ref, b2_ref, Wef_ref, bef_ref,
               out_ref, alp_ref, act_ref):
    ri = ri_ref[...]
    h = jnp.tanh(jnp.dot(ri, W1_ref[...],
                         preferred_element_type=jnp.float32) + b1_ref[...])
    logits = jnp.dot(h, W2_ref[...],
                     preferred_element_type=jnp.float32) + b2_ref[...]
    m = jnp.max(logits, axis=-1, keepdims=True)
    alp = -jnp.log(jnp.sum(jnp.exp(logits - m), axis=-1, keepdims=True))
    col = jax.lax.broadcasted_iota(jnp.int32, logits.shape, 1)
    act = jnp.min(jnp.where(logits == m, col, _E), axis=-1, keepdims=True)
    x = ri[:, :_H]
    y = jnp.dot(x, Wef_ref[...],
                preferred_element_type=jnp.float32) + bef_ref[...]
    acc = jnp.zeros((_BLK, _L), jnp.float32)
    for e in range(_E):
        acc = acc + jnp.where(act == e, y[:, e * _L:(e + 1) * _L], 0.0)
    out_ref[...] = acc
    alp_ref[...] = alp
    act_ref[...] = act


def kernel(hidden_states, typo_vecs, W1, b1, W2, b2, We, be):
    typo_expanded = jnp.broadcast_to(typo_vecs[:, None, :], (_B, _S, _T))
    ri = jnp.concatenate([hidden_states, typo_expanded],
                         axis=-1).reshape(_N, _H + _T)
    Wef = jnp.transpose(We, (1, 0, 2)).reshape(_H, _E * _L)
    bef = be.reshape(1, _E * _L)
    out, alp, act = pl.pallas_call(
        _moe_block,
        grid=(_N // _BLK,),
        in_specs=[
            pl.BlockSpec((_BLK, _H + _T), lambda i: (i, 0)),
            pl.BlockSpec((_H + _T, _DR), lambda i: (0, 0)),
            pl.BlockSpec((1, _DR), lambda i: (0, 0)),
            pl.BlockSpec((_DR, _E), lambda i: (0, 0)),
            pl.BlockSpec((1, _E), lambda i: (0, 0)),
            pl.BlockSpec((_H, _E * _L), lambda i: (0, 0)),
            pl.BlockSpec((1, _E * _L), lambda i: (0, 0)),
        ],
        out_specs=[
            pl.BlockSpec((_BLK, _L), lambda i: (i, 0)),
            pl.BlockSpec((_BLK, 1), lambda i: (i, 0)),
            pl.BlockSpec((_BLK, 1), lambda i: (i, 0)),
        ],
        out_shape=[
            jax.ShapeDtypeStruct((_N, _L), jnp.float32),
            jax.ShapeDtypeStruct((_N, 1), jnp.float32),
            jax.ShapeDtypeStruct((_N, 1), jnp.int32),
        ],
    )(ri, W1, b1.reshape(1, _DR), W2, b2.reshape(1, _E), Wef, bef)
    return (out.reshape(_B, _S, _L),
            alp.reshape(_B, _S),
            act.reshape(_B, _S))


# speedup vs baseline: 24.0842x; 1.1977x over previous
import jax
import jax.numpy as jnp
from jax.experimental import pallas as pl

_B, _S, _H = 4, 2048, 768
_T = 65
_E = 8
_L = 128
_DR = 256
_N = _B * _S
_BLK = 512


def _moe_block(hs_ref, typo_ref, W1_ref, b1_ref, W2_ref, b2_ref, We_ref,
               be_ref, out_ref, alp_ref, act_ref):
    x = hs_ref[...]
    t = typo_ref[0]
    ri = jnp.concatenate([x, jnp.broadcast_to(t, (_BLK, _T))], axis=-1)
    h = jnp.tanh(jnp.dot(ri, W1_ref[...],
                         preferred_element_type=jnp.float32) + b1_ref[...])
    logits = jnp.dot(h, W2_ref[...],
                     preferred_element_type=jnp.float32) + b2_ref[...]
    m = jnp.max(logits, axis=-1, keepdims=True)
    alp = -jnp.log(jnp.sum(jnp.exp(logits - m), axis=-1, keepdims=True))
    col = jax.lax.broadcasted_iota(jnp.int32, logits.shape, 1)
    act = jnp.min(jnp.where(logits == m, col, _E), axis=-1, keepdims=True)
    acc = jnp.zeros((_BLK, _L), jnp.float32)
    for e in range(_E):
        y_e = jnp.dot(x, We_ref[e],
                      preferred_element_type=jnp.float32) + be_ref[e][None, :]
        acc = acc + jnp.where(act == e, y_e, 0.0)
    out_ref[...] = acc
    alp_ref[...] = alp
    act_ref[...] = act


def kernel(hidden_states, typo_vecs, W1, b1, W2, b2, We, be):
    hs = hidden_states.reshape(_N, _H)
    typo3 = typo_vecs.reshape(_B, 1, _T)
    blk_per_batch = _S // _BLK
    out, alp, act = pl.pallas_call(
        _moe_block,
        grid=(_N // _BLK,),
        in_specs=[
            pl.BlockSpec((_BLK, _H), lambda i: (i, 0)),
            pl.BlockSpec((1, 1, _T), lambda i: (i // blk_per_batch, 0, 0)),
            pl.BlockSpec((_H + _T, _DR), lambda i: (0, 0)),
            pl.BlockSpec((1, _DR), lambda i: (0, 0)),
            pl.BlockSpec((_DR, _E), lambda i: (0, 0)),
            pl.BlockSpec((1, _E), lambda i: (0, 0)),
            pl.BlockSpec((_E, _H, _L), lambda i: (0, 0, 0)),
            pl.BlockSpec((_E, _L), lambda i: (0, 0)),
        ],
        out_specs=[
            pl.BlockSpec((_BLK, _L), lambda i: (i, 0)),
            pl.BlockSpec((_BLK, 1), lambda i: (i, 0)),
            pl.BlockSpec((_BLK, 1), lambda i: (i, 0)),
        ],
        out_shape=[
            jax.ShapeDtypeStruct((_N, _L), jnp.float32),
            jax.ShapeDtypeStruct((_N, 1), jnp.float32),
            jax.ShapeDtypeStruct((_N, 1), jnp.int32),
        ],
    )(hs, typo3, W1, b1.reshape(1, _DR), W2, b2.reshape(1, _E), We, be)
    return (out.reshape(_B, _S, _L),
            alp.reshape(_B, _S),
            act.reshape(_B, _S))


# speedup vs baseline: 28.6685x; 1.1903x over previous
import jax
import jax.numpy as jnp
from jax.experimental import pallas as pl

_B, _S, _H = 4, 2048, 768
_T = 65
_E = 8
_L = 128
_DR = 256
_N = _B * _S
_BLK = 512


def _moe_block(hs_ref, typo_ref, W1_ref, b1_ref, W2_ref, b2_ref, Wef_ref,
               bef_ref, out_ref, alp_ref, act_ref):
    x = hs_ref[...]
    t = typo_ref[0]
    ri = jnp.concatenate([x, jnp.broadcast_to(t, (_BLK, _T))], axis=-1)
    h = jnp.tanh(jnp.dot(ri, W1_ref[...],
                         preferred_element_type=jnp.float32) + b1_ref[...])
    logits = jnp.dot(h, W2_ref[...],
                     preferred_element_type=jnp.float32) + b2_ref[...]
    m = jnp.max(logits, axis=-1, keepdims=True)
    alp = -jnp.log(jnp.sum(jnp.exp(logits - m), axis=-1, keepdims=True))
    col = jax.lax.broadcasted_iota(jnp.int32, logits.shape, 1)
    act = jnp.min(jnp.where(logits == m, col, _E), axis=-1, keepdims=True)
    y = jnp.dot(x, Wef_ref[...],
                preferred_element_type=jnp.float32) + bef_ref[...]
    acc = jnp.zeros((_BLK, _L), jnp.float32)
    for e in range(_E):
        acc = acc + jnp.where(act == e, y[:, e * _L:(e + 1) * _L], 0.0)
    out_ref[...] = acc
    alp_ref[...] = alp
    act_ref[...] = act


def kernel(hidden_states, typo_vecs, W1, b1, W2, b2, We, be):
    hs = hidden_states.reshape(_N, _H)
    typo3 = typo_vecs.reshape(_B, 1, _T)
    blk_per_batch = _S // _BLK
    out, alp, act = pl.pallas_call(
        _moe_block,
        grid=(_N // _BLK,),
        in_specs=[
            pl.BlockSpec((_BLK, _H), lambda i: (i, 0)),
            pl.BlockSpec((1, 1, _T), lambda i: (i // blk_per_batch, 0, 0)),
            pl.BlockSpec((_H + _T, _DR), lambda i: (0, 0)),
            pl.BlockSpec((1, _DR), lambda i: (0, 0)),
            pl.BlockSpec((_DR, _E), lambda i: (0, 0)),
            pl.BlockSpec((1, _E), lambda i: (0, 0)),
            pl.BlockSpec((_H, _E * _L), lambda i: (0, 0)),
            pl.BlockSpec((1, _E * _L), lambda i: (0, 0)),
        ],
        out_specs=[
            pl.BlockSpec((_BLK, _L), lambda i: (i, 0)),
            pl.BlockSpec((_BLK, 1), lambda i: (i, 0)),
            pl.BlockSpec((_BLK, 1), lambda i: (i, 0)),
        ],
        out_shape=[
            jax.ShapeDtypeStruct((_N, _L), jnp.float32),
            jax.ShapeDtypeStruct((_N, 1), jnp.float32),
            jax.ShapeDtypeStruct((_N, 1), jnp.int32),
        ],
    )(hs, typo3, W1, b1.reshape(1, _DR), W2, b2.reshape(1, _E),
      jnp.transpose(We, (1, 0, 2)).reshape(_H, _E * _L),
      be.reshape(1, _E * _L))
    return (out.reshape(_B, _S, _L),
            alp.reshape(_B, _S),
            act.reshape(_B, _S))


# speedup vs baseline: 31.1093x; 1.0851x over previous
import jax
import jax.numpy as jnp
from jax.experimental import pallas as pl

_B, _S, _H = 4, 2048, 768
_T = 65
_E = 8
_L = 128
_DR = 256
_N = _B * _S
_BLK = 1024


def _moe_block(hs_ref, typo_ref, W1_ref, b1_ref, W2_ref, b2_ref, Wef_ref,
               bef_ref, out_ref, alp_ref, act_ref):
    x = hs_ref[...]
    t = typo_ref[0]
    ri = jnp.concatenate([x, jnp.broadcast_to(t, (_BLK, _T))], axis=-1)
    h = jnp.tanh(jnp.dot(ri, W1_ref[...],
                         preferred_element_type=jnp.float32) + b1_ref[...])
    logits = jnp.dot(h, W2_ref[...],
                     preferred_element_type=jnp.float32) + b2_ref[...]
    m = jnp.max(logits, axis=-1, keepdims=True)
    alp = -jnp.log(jnp.sum(jnp.exp(logits - m), axis=-1, keepdims=True))
    col = jax.lax.broadcasted_iota(jnp.int32, logits.shape, 1)
    act = jnp.min(jnp.where(logits == m, col, _E), axis=-1, keepdims=True)
    y = jnp.dot(x, Wef_ref[...],
                preferred_element_type=jnp.float32) + bef_ref[...]
    acc = jnp.zeros((_BLK, _L), jnp.float32)
    for e in range(_E):
        acc = acc + jnp.where(act == e, y[:, e * _L:(e + 1) * _L], 0.0)
    out_ref[...] = acc
    alp_ref[...] = alp
    act_ref[...] = act


def kernel(hidden_states, typo_vecs, W1, b1, W2, b2, We, be):
    hs = hidden_states.reshape(_N, _H)
    typo3 = typo_vecs.reshape(_B, 1, _T)
    blk_per_batch = _S // _BLK
    out, alp, act = pl.pallas_call(
        _moe_block,
        grid=(_N // _BLK,),
        in_specs=[
            pl.BlockSpec((_BLK, _H), lambda i: (i, 0)),
            pl.BlockSpec((1, 1, _T), lambda i: (i // blk_per_batch, 0, 0)),
            pl.BlockSpec((_H + _T, _DR), lambda i: (0, 0)),
            pl.BlockSpec((1, _DR), lambda i: (0, 0)),
            pl.BlockSpec((_DR, _E), lambda i: (0, 0)),
            pl.BlockSpec((1, _E), lambda i: (0, 0)),
            pl.BlockSpec((_H, _E * _L), lambda i: (0, 0)),
            pl.BlockSpec((1, _E * _L), lambda i: (0, 0)),
        ],
        out_specs=[
            pl.BlockSpec((_BLK, _L), lambda i: (i, 0)),
            pl.BlockSpec((_BLK, 1), lambda i: (i, 0)),
            pl.BlockSpec((_BLK, 1), lambda i: (i, 0)),
        ],
        out_shape=[
            jax.ShapeDtypeStruct((_N, _L), jnp.float32),
            jax.ShapeDtypeStruct((_N, 1), jnp.float32),
            jax.ShapeDtypeStruct((_N, 1), jnp.int32),
        ],
    )(hs, typo3, W1, b1.reshape(1, _DR), W2, b2.reshape(1, _E),
      jnp.transpose(We, (1, 0, 2)).reshape(_H, _E * _L),
      be.reshape(1, _E * _L))
    return (out.reshape(_B, _S, _L),
            alp.reshape(_B, _S),
            act.reshape(_B, _S))


# speedup vs baseline: 31.8424x; 1.0236x over previous
import jax
import jax.numpy as jnp
from jax.experimental import pallas as pl

_B, _S, _H = 4, 2048, 768
_T = 65
_E = 8
_L = 128
_DR = 256
_N = _B * _S
_BLK = 2048


def _moe_block(hs_ref, typo_ref, W1_ref, b1_ref, W2_ref, b2_ref, Wef_ref,
               bef_ref, out_ref, alp_ref, act_ref):
    x = hs_ref[...]
    t = typo_ref[0]
    ri = jnp.concatenate([x, jnp.broadcast_to(t, (_BLK, _T))], axis=-1)
    h = jnp.tanh(jnp.dot(ri, W1_ref[...],
                         preferred_element_type=jnp.float32) + b1_ref[...])
    logits = jnp.dot(h, W2_ref[...],
                     preferred_element_type=jnp.float32) + b2_ref[...]
    m = jnp.max(logits, axis=-1, keepdims=True)
    alp = -jnp.log(jnp.sum(jnp.exp(logits - m), axis=-1, keepdims=True))
    col = jax.lax.broadcasted_iota(jnp.int32, logits.shape, 1)
    act = jnp.min(jnp.where(logits == m, col, _E), axis=-1, keepdims=True)
    y = jnp.dot(x, Wef_ref[...],
                preferred_element_type=jnp.float32) + bef_ref[...]
    acc = jnp.zeros((_BLK, _L), jnp.float32)
    for e in range(_E):
        acc = acc + jnp.where(act == e, y[:, e * _L:(e + 1) * _L], 0.0)
    out_ref[...] = acc
    alp_ref[...] = alp
    act_ref[...] = act


def kernel(hidden_states, typo_vecs, W1, b1, W2, b2, We, be):
    hs = hidden_states.reshape(_N, _H)
    typo3 = typo_vecs.reshape(_B, 1, _T)
    blk_per_batch = _S // _BLK
    out, alp, act = pl.pallas_call(
        _moe_block,
        grid=(_N // _BLK,),
        in_specs=[
            pl.BlockSpec((_BLK, _H), lambda i: (i, 0)),
            pl.BlockSpec((1, 1, _T), lambda i: (i // blk_per_batch, 0, 0)),
            pl.BlockSpec((_H + _T, _DR), lambda i: (0, 0)),
            pl.BlockSpec((1, _DR), lambda i: (0, 0)),
            pl.BlockSpec((_DR, _E), lambda i: (0, 0)),
            pl.BlockSpec((1, _E), lambda i: (0, 0)),
            pl.BlockSpec((_H, _E * _L), lambda i: (0, 0)),
            pl.BlockSpec((1, _E * _L), lambda i: (0, 0)),
        ],
        out_specs=[
            pl.BlockSpec((_BLK, _L), lambda i: (i, 0)),
            pl.BlockSpec((_BLK, 1), lambda i: (i, 0)),
            pl.BlockSpec((_BLK, 1), lambda i: (i, 0)),
        ],
        out_shape=[
            jax.ShapeDtypeStruct((_N, _L), jnp.float32),
            jax.ShapeDtypeStruct((_N, 1), jnp.float32),
            jax.ShapeDtypeStruct((_N, 1), jnp.int32),
        ],
    )(hs, typo3, W1, b1.reshape(1, _DR), W2, b2.reshape(1, _E),
      jnp.transpose(We, (1, 0, 2)).reshape(_H, _E * _L),
      be.reshape(1, _E * _L))
    return (out.reshape(_B, _S, _L),
            alp.reshape(_B, _S),
            act.reshape(_B, _S))


# speedup vs baseline: 33.0851x; 1.0390x over previous
import jax
import jax.numpy as jnp
from jax.experimental import pallas as pl

_B, _S, _H = 4, 2048, 768
_T = 65
_E = 8
_L = 128
_DR = 256
_N = _B * _S
_BLK = 2048


def _moe_block(hs_ref, typo_ref, W1_ref, b1_ref, W2_ref, b2_ref, Wef_ref,
               bef_ref, out_ref, alp_ref, act_ref):
    x = hs_ref[...]
    t = typo_ref[0]
    ri = jnp.concatenate([x, jnp.broadcast_to(t, (_BLK, _T))], axis=-1)
    h = jnp.tanh(jnp.dot(ri, W1_ref[...],
                         preferred_element_type=jnp.float32) + b1_ref[...])
    logits = jnp.dot(h, W2_ref[...],
                     preferred_element_type=jnp.float32) + b2_ref[...]
    m = jnp.max(logits, axis=-1, keepdims=True)
    alp = -jnp.log(jnp.sum(jnp.exp(logits - m), axis=-1, keepdims=True))
    col = jax.lax.broadcasted_iota(jnp.int32, logits.shape, 1)
    act = jnp.min(jnp.where(logits == m, col, _E), axis=-1, keepdims=True)
    y = jnp.dot(x.astype(jnp.bfloat16), Wef_ref[...],
                preferred_element_type=jnp.float32) + bef_ref[...]
    acc = jnp.zeros((_BLK, _L), jnp.float32)
    for e in range(_E):
        acc = acc + jnp.where(act == e, y[:, e * _L:(e + 1) * _L], 0.0)
    out_ref[...] = acc
    alp_ref[...] = alp
    act_ref[...] = act


def kernel(hidden_states, typo_vecs, W1, b1, W2, b2, We, be):
    hs = hidden_states.reshape(_N, _H)
    typo3 = typo_vecs.reshape(_B, 1, _T)
    blk_per_batch = _S // _BLK
    out, alp, act = pl.pallas_call(
        _moe_block,
        grid=(_N // _BLK,),
        in_specs=[
            pl.BlockSpec((_BLK, _H), lambda i: (i, 0)),
            pl.BlockSpec((1, 1, _T), lambda i: (i // blk_per_batch, 0, 0)),
            pl.BlockSpec((_H + _T, _DR), lambda i: (0, 0)),
            pl.BlockSpec((1, _DR), lambda i: (0, 0)),
            pl.BlockSpec((_DR, _E), lambda i: (0, 0)),
            pl.BlockSpec((1, _E), lambda i: (0, 0)),
            pl.BlockSpec((_H, _E * _L), lambda i: (0, 0)),
            pl.BlockSpec((1, _E * _L), lambda i: (0, 0)),
        ],
        out_specs=[
            pl.BlockSpec((_BLK, _L), lambda i: (i, 0)),
            pl.BlockSpec((_BLK, 1), lambda i: (i, 0)),
            pl.BlockSpec((_BLK, 1), lambda i: (i, 0)),
        ],
        out_shape=[
            jax.ShapeDtypeStruct((_N, _L), jnp.float32),
            jax.ShapeDtypeStruct((_N, 1), jnp.float32),
            jax.ShapeDtypeStruct((_N, 1), jnp.int32),
        ],
    )(hs, typo3, W1, b1.reshape(1, _DR), W2, b2.reshape(1, _E),
      jnp.transpose(We, (1, 0, 2)).reshape(_H, _E * _L).astype(jnp.bfloat16),
      be.reshape(1, _E * _L))
    return (out.reshape(_B, _S, _L),
            alp.reshape(_B, _S),
            act.reshape(_B, _S))
